# baseline (speedup 1.0000x reference)
import functools

import jax
import jax.numpy as jnp
from jax import lax
from jax.experimental import pallas as pl
from jax.experimental.pallas import tpu as pltpu
from jax.experimental.pallas import tpu_sc as plsc

NUM_CORES = 2
NUM_SUBCORES = 16
NUM_WORKERS = NUM_CORES * NUM_SUBCORES
LANES = 16

CHUNK_ROWS = 16
N_PHASES = 2


def _sc_encode(idx2d, val2d, w_pairs, bias, *, B, S, K, D):
    N = B * S
    SK = S * K
    V = w_pairs.shape[0]
    Dp = D // 2
    half_p = Dp // 2
    n_rgrp = NUM_WORKERS // 2
    rows_per_w = N // n_rgrp
    b_per_w = B // n_rgrp
    b_per_phase = b_per_w // N_PHASES
    rows_per_phase = rows_per_w // N_PHASES
    chunk_lk = CHUNK_ROWS * K
    chunks_per_b = S // CHUNK_ROWS
    n_chunks = rows_per_phase // CHUNK_ROWS

    mesh = plsc.VectorSubcoreMesh(
        core_axis_name="c", subcore_axis_name="s",
        num_cores=NUM_CORES, num_subcores=NUM_SUBCORES)

    @functools.partial(
        pl.kernel,
        out_type=jax.ShapeDtypeStruct((B, S, D), jnp.float32),
        mesh=mesh,
        compiler_params=pltpu.CompilerParams(
            needs_layout_passes=False, use_tc_tiling_on_sc=False),
        scratch_types=[
            pltpu.VMEM((V, half_p), jnp.int32),
            pltpu.VMEM((b_per_phase, SK), jnp.int32),
            pltpu.VMEM((b_per_phase, SK), jnp.float32),
            pltpu.VMEM((CHUNK_ROWS, D // 2), jnp.float32),
            pltpu.VMEM((CHUNK_ROWS, D // 2), jnp.float32),
            pltpu.VMEM((D,), jnp.float32),
            pltpu.SemaphoreType.DMA,
            pltpu.SemaphoreType.DMA,
            pltpu.SemaphoreType.DMA,
        ],
    )
    def sc_kernel(idx_hbm, val_hbm, w_hbm, bias_hbm, out_hbm,
                  table_v, idx_v, val_v, out_a, out_b, bias_v,
                  sem, ssem_a, ssem_b):
        wid = lax.axis_index("s") * NUM_CORES + lax.axis_index("c")
        rgrp = wid // 2
        half = wid % 2
        base_b = rgrp * b_per_w
        col0 = half * (D // 2)

        pltpu.sync_copy(w_hbm.at[:, pl.ds(half * half_p, half_p)], table_v)
        pltpu.sync_copy(bias_hbm, bias_v)

        bias_regs = [
            plsc.pack(bias_v[pl.ds(col0 + 2 * LANES * j, LANES)],
                      bias_v[pl.ds(col0 + 2 * LANES * j + LANES, LANES)],
                      format=plsc.PackFormat.INTERLEAVED)
            for j in range(half_p // LANES)
        ]
        col_iotas = [lax.iota(jnp.int32, LANES) + LANES * j
                     for j in range(half_p // LANES)]

        def compute(br, col, rows_v_unused, out_v):
            del rows_v_unused
            for r in range(CHUNK_ROWS):
                vv = val_v[br, pl.ds(col + (r // 2) * LANES, LANES)]
                iv = idx_v[br, pl.ds(col + (r // 2) * LANES, LANES)]
                vbs, rows16 = [], []
                for k in range(K):
                    lane = jnp.full((LANES,), (r % 2) * K + k, jnp.int32)
                    v = jnp.take_along_axis(vv, lane, axis=0,
                                            mode="promise_in_bounds")
                    vbs.append(
                        plsc.pack(v, v, format=plsc.PackFormat.INTERLEAVED))
                    rows16.append(
                        jnp.take_along_axis(iv, lane, axis=0,
                                            mode="promise_in_bounds"))
                for j in range(half_p // LANES):
                    p = []
                    for k in range(K):
                        u = plsc.load_gather(
                            table_v, [rows16[k], col_iotas[j]])
                        wv = plsc.bitcast(u, jnp.bfloat16)
                        p.append(wv * vbs[k])
                    acc = (((p[0] + p[1]) + (p[2] + p[3]))
                           + ((p[4] + p[5]) + (p[6] + p[7]))) + bias_regs[j]
                    a_f32, b_f32 = plsc.unpack(
                        acc, format=plsc.PackFormat.INTERLEAVED)
                    out_v[r, pl.ds(2 * LANES * j, LANES)] = a_f32
                    out_v[r, pl.ds(2 * LANES * j + LANES, LANES)] = b_f32

        def out_slice(ph, c):
            b = base_b + ph * b_per_phase + c // chunks_per_b
            return out_hbm.at[b, pl.ds((c % chunks_per_b) * CHUNK_ROWS,
                                       CHUNK_ROWS), pl.ds(col0, D // 2)]

        for ph in range(N_PHASES):
            b0 = base_b + ph * b_per_phase
            pltpu.sync_copy(idx_hbm.at[pl.ds(b0, b_per_phase), :], idx_v)
            pltpu.sync_copy(val_hbm.at[pl.ds(b0, b_per_phase), :], val_v)

            @pl.loop(0, n_chunks, step=2)
            def chunk_pair(c):
                br = c // chunks_per_b
                col = (c % chunks_per_b) * chunk_lk

                @pl.when(c >= 2)
                def _():
                    pltpu.make_async_copy(
                        out_a, out_slice(ph, c - 2), ssem_a).wait()

                compute(br, col, None, out_a)
                pltpu.async_copy(out_a, out_slice(ph, c), ssem_a)

                br1 = (c + 1) // chunks_per_b
                col1 = ((c + 1) % chunks_per_b) * chunk_lk

                @pl.when(c >= 2)
                def _():
                    pltpu.make_async_copy(
                        out_b, out_slice(ph, c - 1), ssem_b).wait()

                compute(br1, col1, None, out_b)
                pltpu.async_copy(out_b, out_slice(ph, c + 1), ssem_b)

            pltpu.make_async_copy(
                out_a, out_slice(ph, n_chunks - 2), ssem_a).wait()
            pltpu.make_async_copy(
                out_b, out_slice(ph, n_chunks - 1), ssem_b).wait()

    return sc_kernel(idx2d, val2d, w_pairs, bias)


def _prep_weight(weight):
    V, D = weight.shape
    wb = weight.astype(jnp.bfloat16).reshape(V, D // 32, 2, 16)
    wb = wb.transpose(0, 1, 3, 2)
    return lax.bitcast_convert_type(wb, jnp.int32).reshape(V, D // 2)


def kernel(piece_indices, piece_values, weight, bias):
    B, S, K = piece_indices.shape
    D = weight.shape[1]
    idx2d = piece_indices.reshape(B, S * K)
    val2d = piece_values.reshape(B, S * K)
    w_pairs = _prep_weight(weight)
    return _sc_encode(idx2d, val2d, w_pairs, bias, B=B, S=S, K=K, D=D)


# speedup vs baseline: 3.5579x; 3.5579x over previous
import functools

import jax
import jax.numpy as jnp
from jax import lax
from jax.experimental import pallas as pl
from jax.experimental.pallas import tpu as pltpu
from jax.experimental.pallas import tpu_sc as plsc

NUM_CORES = 2
NUM_SUBCORES = 16
NUM_WORKERS = NUM_CORES * NUM_SUBCORES
LANES = 16

CHUNK_ROWS = 16


def _sc_encode(idx2d, val2d, w_bf16, bias, *, B, S, K, D):
    N = B * S
    SK = S * K
    rows_per_w = N // NUM_WORKERS
    b_per_w = B // NUM_WORKERS
    lk_per_w = rows_per_w * K
    chunk_lk = CHUNK_ROWS * K
    n_chunks = rows_per_w // CHUNK_ROWS
    chunks_per_b = S // CHUNK_ROWS
    p_vecs = D // (2 * LANES)

    mesh = plsc.VectorSubcoreMesh(
        core_axis_name="c", subcore_axis_name="s",
        num_cores=NUM_CORES, num_subcores=NUM_SUBCORES)

    @functools.partial(
        pl.kernel,
        out_type=jax.ShapeDtypeStruct((B, S, D), jnp.float32),
        mesh=mesh,
        compiler_params=pltpu.CompilerParams(
            needs_layout_passes=False, use_tc_tiling_on_sc=False),
        scratch_types=[
            pltpu.VMEM((b_per_w, SK), jnp.int32),
            pltpu.VMEM((b_per_w, SK), jnp.float32),
            pltpu.VMEM((chunk_lk, D), jnp.bfloat16),
            pltpu.VMEM((chunk_lk, D), jnp.bfloat16),
            pltpu.VMEM((CHUNK_ROWS, D), jnp.float32),
            pltpu.VMEM((CHUNK_ROWS, D), jnp.float32),
            pltpu.VMEM((D,), jnp.float32),
            pltpu.VMEM_SHARED((2496, D), jnp.bfloat16),
            pltpu.SemaphoreType.DMA,
            pltpu.SemaphoreType.DMA,
            pltpu.SemaphoreType.DMA,
            pltpu.SemaphoreType.DMA,
        ],
    )
    def sc_kernel(idx_hbm, val_hbm, w_hbm, bias_hbm, out_hbm,
                  idx_v, val_v, rows_a, rows_b, out_a, out_b, bias_v,
                  w_shared, gsem_a, gsem_b, ssem_a, ssem_b):
        wid = lax.axis_index("s") * NUM_CORES + lax.axis_index("c")
        base_b = wid * b_per_w

        @pl.when(lax.axis_index("s") == 0)
        def _():
            pltpu.sync_copy(w_hbm, w_shared)

        pltpu.sync_copy(idx_hbm.at[pl.ds(base_b, b_per_w), :], idx_v)
        pltpu.sync_copy(val_hbm.at[pl.ds(base_b, b_per_w), :], val_v)
        pltpu.sync_copy(bias_hbm, bias_v)
        plsc.subcore_barrier()

        bias_regs = [
            plsc.pack(bias_v[pl.ds(2 * LANES * j, LANES)],
                      bias_v[pl.ds(2 * LANES * j + LANES, LANES)],
                      format=plsc.PackFormat.INTERLEAVED)
            for j in range(p_vecs)
        ]

        def chunk_pos(c):
            return c // chunks_per_b, (c % chunks_per_b) * chunk_lk

        def gather(c, rows, sem):
            br, col = chunk_pos(c)
            pltpu.async_copy(
                w_shared.at[idx_v.at[br, pl.ds(col, chunk_lk)]], rows, sem)

        def gather_wait(c, rows, sem):
            br, col = chunk_pos(c)
            pltpu.make_async_copy(
                w_shared.at[idx_v.at[br, pl.ds(col, chunk_lk)]], rows,
                sem).wait()

        def out_slice(c):
            return out_hbm.at[base_b + c // chunks_per_b,
                              pl.ds((c % chunks_per_b) * CHUNK_ROWS,
                                    CHUNK_ROWS), :]

        def compute(c, rows_v, out_v):
            br, col = chunk_pos(c)
            for r in range(CHUNK_ROWS):
                vv = val_v[br, pl.ds(col + (r // 2) * LANES, LANES)]
                vbs = []
                for k in range(K):
                    lane = jnp.full((LANES,), (r % 2) * K + k, jnp.int32)
                    v = jnp.take_along_axis(vv, lane, axis=0,
                                            mode="promise_in_bounds")
                    vbs.append(
                        plsc.pack(v, v, format=plsc.PackFormat.INTERLEAVED))
                for j in range(p_vecs):
                    p = [rows_v[r * K + k, pl.ds(2 * LANES * j, 2 * LANES)]
                         * vbs[k] for k in range(K)]
                    acc = (((p[0] + p[1]) + (p[2] + p[3]))
                           + ((p[4] + p[5]) + (p[6] + p[7]))) + bias_regs[j]
                    a_f32, b_f32 = plsc.unpack(
                        acc, format=plsc.PackFormat.INTERLEAVED)
                    out_v[r, pl.ds(2 * LANES * j, LANES)] = a_f32
                    out_v[r, pl.ds(2 * LANES * j + LANES, LANES)] = b_f32

        gather(0, rows_a, gsem_a)

        @pl.loop(0, n_chunks, step=2)
        def chunk_pair(c):
            gather(c + 1, rows_b, gsem_b)
            gather_wait(c, rows_a, gsem_a)

            @pl.when(c >= 2)
            def _():
                pltpu.make_async_copy(out_a, out_slice(c - 2), ssem_a).wait()

            compute(c, rows_a, out_a)
            pltpu.async_copy(out_a, out_slice(c), ssem_a)

            @pl.when(c + 2 < n_chunks)
            def _():
                gather(c + 2, rows_a, gsem_a)

            gather_wait(c + 1, rows_b, gsem_b)

            @pl.when(c >= 2)
            def _():
                pltpu.make_async_copy(out_b, out_slice(c - 1), ssem_b).wait()

            compute(c + 1, rows_b, out_b)
            pltpu.async_copy(out_b, out_slice(c + 1), ssem_b)

        pltpu.make_async_copy(out_a, out_slice(n_chunks - 2), ssem_a).wait()
        pltpu.make_async_copy(out_b, out_slice(n_chunks - 1), ssem_b).wait()

    return sc_kernel(idx2d, val2d, w_bf16, bias)


def _prep_weight(weight):
    V, D = weight.shape
    wb = weight.astype(jnp.bfloat16).reshape(V, D // 32, 2, 16)
    return wb.transpose(0, 1, 3, 2).reshape(V, D)


def kernel(piece_indices, piece_values, weight, bias):
    B, S, K = piece_indices.shape
    D = weight.shape[1]
    idx2d = piece_indices.reshape(B, S * K)
    val2d = piece_values.reshape(B, S * K)
    w_bf16 = _prep_weight(weight)
    return _sc_encode(idx2d, val2d, w_bf16, bias, B=B, S=S, K=K, D=D)


# speedup vs baseline: 3.6094x; 1.0145x over previous
import functools

import jax
import jax.numpy as jnp
from jax import lax
from jax.experimental import pallas as pl
from jax.experimental.pallas import tpu as pltpu
from jax.experimental.pallas import tpu_sc as plsc

NUM_CORES = 2
NUM_SUBCORES = 16
NUM_WORKERS = NUM_CORES * NUM_SUBCORES
LANES = 16

CHUNK_ROWS = 16


def _sc_encode(idx2d, val2d, w_f32, bias, *, B, S, K, D):
    N = B * S
    SK = S * K
    V = w_f32.shape[0]
    rows_per_w = N // NUM_WORKERS
    b_per_w = B // NUM_WORKERS
    lk_per_w = rows_per_w * K
    chunk_lk = CHUNK_ROWS * K
    n_chunks = rows_per_w // CHUNK_ROWS
    chunks_per_b = S // CHUNK_ROWS
    p_vecs = D // (2 * LANES)

    mesh = plsc.VectorSubcoreMesh(
        core_axis_name="c", subcore_axis_name="s",
        num_cores=NUM_CORES, num_subcores=NUM_SUBCORES)

    @functools.partial(
        pl.kernel,
        out_type=jax.ShapeDtypeStruct((B, S, D), jnp.float32),
        mesh=mesh,
        compiler_params=pltpu.CompilerParams(
            needs_layout_passes=False, use_tc_tiling_on_sc=False),
        scratch_types=[
            pltpu.VMEM((b_per_w, SK), jnp.int32),
            pltpu.VMEM((b_per_w, SK), jnp.float32),
            pltpu.VMEM((chunk_lk, D), jnp.bfloat16),
            pltpu.VMEM((chunk_lk, D), jnp.bfloat16),
            pltpu.VMEM((CHUNK_ROWS, D), jnp.float32),
            pltpu.VMEM((CHUNK_ROWS, D), jnp.float32),
            pltpu.VMEM((D,), jnp.float32),
            pltpu.VMEM_SHARED((V, D), jnp.bfloat16),
            pltpu.VMEM((V // NUM_SUBCORES, D), jnp.float32),
            pltpu.VMEM((V // NUM_SUBCORES, D), jnp.bfloat16),
            pltpu.SemaphoreType.DMA,
            pltpu.SemaphoreType.DMA,
            pltpu.SemaphoreType.DMA,
            pltpu.SemaphoreType.DMA,
        ],
    )
    def sc_kernel(idx_hbm, val_hbm, w_hbm, bias_hbm, out_hbm,
                  idx_v, val_v, rows_a, rows_b, out_a, out_b, bias_v,
                  w_shared, wst_v, pst_v, gsem_a, gsem_b, ssem_a, ssem_b):
        sid = lax.axis_index("s")
        wid = sid * NUM_CORES + lax.axis_index("c")
        base_b = wid * b_per_w
        v_rows = V // NUM_SUBCORES

        pltpu.sync_copy(w_hbm.at[pl.ds(sid * v_rows, v_rows), :], wst_v)

        @pl.loop(0, v_rows)
        def pack_row(t):
            for j in range(p_vecs):
                packed = plsc.pack(
                    wst_v[t, pl.ds(2 * LANES * j, LANES)],
                    wst_v[t, pl.ds(2 * LANES * j + LANES, LANES)],
                    format=plsc.PackFormat.INTERLEAVED)
                pst_v[t, pl.ds(2 * LANES * j, 2 * LANES)] = packed

        pltpu.sync_copy(pst_v, w_shared.at[pl.ds(sid * v_rows, v_rows), :])

        pltpu.sync_copy(idx_hbm.at[pl.ds(base_b, b_per_w), :], idx_v)
        pltpu.sync_copy(val_hbm.at[pl.ds(base_b, b_per_w), :], val_v)
        pltpu.sync_copy(bias_hbm, bias_v)
        plsc.subcore_barrier()

        bias_regs = [
            plsc.pack(bias_v[pl.ds(2 * LANES * j, LANES)],
                      bias_v[pl.ds(2 * LANES * j + LANES, LANES)],
                      format=plsc.PackFormat.INTERLEAVED)
            for j in range(p_vecs)
        ]

        lane_consts = [jnp.full((LANES,), i, jnp.int32)
                       for i in range(LANES)]

        def chunk_pos(c):
            return c // chunks_per_b, (c % chunks_per_b) * chunk_lk

        def gather(c, rows, sem):
            br, col = chunk_pos(c)
            pltpu.async_copy(
                w_shared.at[idx_v.at[br, pl.ds(col, chunk_lk)]], rows, sem)

        def gather_wait(c, rows, sem):
            br, col = chunk_pos(c)
            pltpu.make_async_copy(
                w_shared.at[idx_v.at[br, pl.ds(col, chunk_lk)]], rows,
                sem).wait()

        def out_slice(c):
            return out_hbm.at[base_b + c // chunks_per_b,
                              pl.ds((c % chunks_per_b) * CHUNK_ROWS,
                                    CHUNK_ROWS), :]

        def compute(c, rows_v, out_v):
            br, col = chunk_pos(c)
            for r in range(CHUNK_ROWS):
                vv = val_v[br, pl.ds(col + (r // 2) * LANES, LANES)]
                vbs = []
                for k in range(K):
                    lane = lane_consts[(r % 2) * K + k]
                    v = jnp.take_along_axis(vv, lane, axis=0,
                                            mode="promise_in_bounds")
                    vbs.append(
                        plsc.pack(v, v, format=plsc.PackFormat.INTERLEAVED))
                for j in range(p_vecs):
                    p = [rows_v[r * K + k, pl.ds(2 * LANES * j, 2 * LANES)]
                         * vbs[k] for k in range(K)]
                    acc = (((p[0] + p[1]) + (p[2] + p[3]))
                           + ((p[4] + p[5]) + (p[6] + p[7]))) + bias_regs[j]
                    a_f32, b_f32 = plsc.unpack(
                        acc, format=plsc.PackFormat.INTERLEAVED)
                    out_v[r, pl.ds(2 * LANES * j, LANES)] = a_f32
                    out_v[r, pl.ds(2 * LANES * j + LANES, LANES)] = b_f32

        gather(0, rows_a, gsem_a)

        @pl.loop(0, n_chunks, step=2)
        def chunk_pair(c):
            gather(c + 1, rows_b, gsem_b)
            gather_wait(c, rows_a, gsem_a)

            @pl.when(c >= 2)
            def _():
                pltpu.make_async_copy(out_a, out_slice(c - 2), ssem_a).wait()

            compute(c, rows_a, out_a)
            pltpu.async_copy(out_a, out_slice(c), ssem_a)

            @pl.when(c + 2 < n_chunks)
            def _():
                gather(c + 2, rows_a, gsem_a)

            gather_wait(c + 1, rows_b, gsem_b)

            @pl.when(c >= 2)
            def _():
                pltpu.make_async_copy(out_b, out_slice(c - 1), ssem_b).wait()

            compute(c + 1, rows_b, out_b)
            pltpu.async_copy(out_b, out_slice(c + 1), ssem_b)

        pltpu.make_async_copy(out_a, out_slice(n_chunks - 2), ssem_a).wait()
        pltpu.make_async_copy(out_b, out_slice(n_chunks - 1), ssem_b).wait()

    return sc_kernel(idx2d, val2d, w_f32, bias)


def kernel(piece_indices, piece_values, weight, bias):
    B, S, K = piece_indices.shape
    D = weight.shape[1]
    idx2d = piece_indices.reshape(B, S * K)
    val2d = piece_values.reshape(B, S * K)
    return _sc_encode(idx2d, val2d, weight, bias, B=B, S=S, K=K, D=D)
